# baseline (speedup 1.0000x reference)
import functools

import jax
import jax.numpy as jnp
from jax import lax
from jax.experimental import pallas as pl
from jax.experimental.pallas import tpu as pltpu
from jax.experimental.pallas import tpu_sc as plsc

_N = 10000
_E = 320000
_D = 128

_NS = 16
_CHUNK = 125
_EPT = _E // _NS
_NCH = _EPT // _CHUNK
_BS = 32
_NB = _NCH // _BS
_NP = 10240
_RPT = _NP // _NS
_RZ = 128
_RCH = _RPT // _RZ
_CE = 2000
_NCE = _E // _CE
_CV = _CE // 16

_BN = 1024
_NG = _NP // _BN


@functools.lru_cache(maxsize=None)
def _build_seg_kernel(with_counts):
    mesh = plsc.VectorSubcoreMesh(core_axis_name="c", subcore_axis_name="s")

    out_type = [jax.ShapeDtypeStruct((_NP, _D), jnp.float32),
                jax.ShapeDtypeStruct((_NP, _D), jnp.float32)]
    scratch = [
        pltpu.VMEM_SHARED((_NP, _D), jnp.float32),
        pltpu.VMEM((_BS, _CHUNK), jnp.int32),
        pltpu.VMEM((_BS, _CHUNK), jnp.int32),
        pltpu.VMEM((_RZ, _D), jnp.float32),
        pltpu.VMEM((_RZ, _D), jnp.float32),
        pltpu.SemaphoreType.DMA,
        pltpu.SemaphoreType.DMA,
    ]
    if with_counts:
        out_type += [jax.ShapeDtypeStruct((_NP,), jnp.float32),
                     jax.ShapeDtypeStruct((_NP,), jnp.float32)]
        scratch += [
            pltpu.VMEM((_CE,), jnp.int32),
            pltpu.VMEM((_CE,), jnp.int32),
            pltpu.VMEM((_RPT,), jnp.float32),
            pltpu.SemaphoreType.DMA,
            pltpu.SemaphoreType.DMA,
        ]

    def seg(t1, t2, src2d, dst2d, z128, *rest):
        if with_counts:
            (src1d, dst1d, s1_out, s2_out, cin_out, cout_out,
             acc, gidx, sidx, rowbuf, rowbuf1, sem, sem1,
             cidx0, cidx1, partial, csem0, csem1) = rest
        else:
            (s1_out, s2_out,
             acc, gidx, sidx, rowbuf, rowbuf1, sem, sem1) = rest
        cid = lax.axis_index("c")
        tid = lax.axis_index("s")

        def count_scan(e1d, cout_):
            zv = jnp.zeros((16,), jnp.float32)
            for i in range(_RPT // 16):
                partial[pl.ds(16 * i, 16)] = zv
            lo = tid * _RPT
            ones_v = jnp.ones((16,), jnp.float32)
            cbufs = (cidx0, cidx1)
            csems = (csem0, csem1)

            def scan_buf(k):
                for v in range(_CV):
                    idx = cbufs[k][pl.ds(16 * v, 16)]
                    loc = idx - lo
                    m = (loc >= 0) & (loc < _RPT)
                    plsc.addupdate_scatter(partial, [loc], ones_v, mask=m)

            pltpu.async_copy(e1d.at[pl.ds(0, _CE)], cidx0, csem0)

            def pair(p, carry):
                base = pl.multiple_of(p * (2 * _CE), 8)
                pltpu.async_copy(e1d.at[pl.ds(base + _CE, _CE)], cidx1, csem1)
                pltpu.make_async_copy(e1d.at[pl.ds(0, _CE)], cidx0,
                                      csem0).wait()
                scan_buf(0)
                nxt = jnp.minimum(base + 2 * _CE, _E - _CE)
                pltpu.async_copy(e1d.at[pl.ds(nxt, _CE)], cidx0, csem0)
                pltpu.make_async_copy(e1d.at[pl.ds(0, _CE)], cidx1,
                                      csem1).wait()
                scan_buf(1)
                return carry

            lax.fori_loop(0, _NCE // 2, pair, 0)
            pltpu.make_async_copy(e1d.at[pl.ds(0, _CE)], cidx0, csem0).wait()
            pltpu.sync_copy(partial, cout_.at[pl.ds(lo, _RPT)])

        def one_pass(table, g2, s3, out):
            pltpu.sync_copy(z128, rowbuf)
            for z in range(_RCH):
                pltpu.sync_copy(
                    rowbuf, acc.at[pl.ds(tid * _RPT + z * _RZ, _RZ)])
            plsc.subcore_barrier()

            bufs = (rowbuf.at[pl.ds(0, _CHUNK)], rowbuf1.at[pl.ds(0, _CHUNK)])
            sems = (sem, sem1)

            def batch(b, carry):
                r = pl.multiple_of(tid * _NCH + b * _BS, _BS)
                pltpu.sync_copy(g2.at[pl.ds(r, _BS)], gidx)
                pltpu.sync_copy(s3.at[pl.ds(r, _BS)], sidx)
                descs = [None, None]
                descs[0] = pltpu.async_copy(table.at[gidx.at[0]], bufs[0],
                                            sems[0])
                for jj in range(_BS):
                    if jj + 1 < _BS:
                        nb = (jj + 1) % 2
                        descs[nb] = pltpu.async_copy(
                            table.at[gidx.at[jj + 1]], bufs[nb], sems[nb])
                    descs[jj % 2].wait()
                    pltpu.sync_copy(bufs[jj % 2], acc.at[sidx.at[jj]],
                                    add=True)
                return carry

            lax.fori_loop(0, _NB, batch, 0)
            plsc.subcore_barrier()
            for z in range(_RCH):
                r0 = tid * _RPT + z * _RZ
                pltpu.sync_copy(acc.at[pl.ds(r0, _RZ)],
                                out.at[pl.ds(r0, _RZ)])

        @pl.when(cid == 0)
        def _():
            one_pass(t1, src2d, dst2d, s1_out)
            if with_counts:
                count_scan(dst1d, cin_out)

        @pl.when(cid == 1)
        def _():
            one_pass(t2, dst2d, src2d, s2_out)
            if with_counts:
                count_scan(src1d, cout_out)

    params = (pltpu.CompilerParams(needs_layout_passes=False)
              if with_counts else None)
    return pl.kernel(seg, out_type=out_type, mesh=mesh,
                     scratch_types=scratch, compiler_params=params)


def _dot(a, b):
    return jnp.dot(a, b, preferred_element_type=jnp.float32)


def _mm3_body(x_ref, w1_ref, w2_ref, wr_ref, br_ref, t1_ref, t2_ref, r_ref):
    xb = x_ref[...]
    t1_ref[...] = _dot(xb, w1_ref[...])
    t2_ref[...] = _dot(xb, w2_ref[...])
    r_ref[...] = _dot(xb, wr_ref[...]) + br_ref[...]


def _mm3(x, w1, w2, wr, br):
    nd = jax.ShapeDtypeStruct((_N, _D), jnp.float32)
    row = pl.BlockSpec((_BN, _D), lambda i: (i, 0))
    full = pl.BlockSpec((_D, _D), lambda i: (0, 0))
    bias = pl.BlockSpec((1, _D), lambda i: (0, 0))
    return pl.pallas_call(
        _mm3_body,
        grid=(_NG,),
        in_specs=[row, full, full, full, bias],
        out_specs=[row, row, row],
        out_shape=[nd, nd, nd],
    )(x, w1, w2, wr, br.reshape(1, _D))


def _finish(r_ref, s1_ref, s2_ref, cin_ref, cout_ref):
    rin = 1.0 / jnp.maximum(cin_ref[...], 1.0)
    rout = 1.0 / jnp.maximum(cout_ref[...], 1.0)
    return jnp.maximum(
        r_ref[...] + s1_ref[...] * rin + s2_ref[...] * rout, 0.0)


def _mid_body(r_ref, s1_ref, s2_ref, cin_ref, cout_ref,
              w1_ref, w2_ref, wr_ref, br_ref,
              h_ref, t1_ref, t2_ref, r1_ref):
    h = _finish(r_ref, s1_ref, s2_ref, cin_ref, cout_ref)
    h_ref[...] = h
    t1_ref[...] = _dot(h, w1_ref[...])
    t2_ref[...] = _dot(h, w2_ref[...])
    r1_ref[...] = _dot(h, wr_ref[...]) + br_ref[...]


def _mid(r0, s1, s2, cin, cout, w1, w2, wr, br):
    nd = jax.ShapeDtypeStruct((_N, _D), jnp.float32)
    row = pl.BlockSpec((_BN, _D), lambda i: (i, 0))
    full = pl.BlockSpec((_D, _D), lambda i: (0, 0))
    bias = pl.BlockSpec((1, _D), lambda i: (0, 0))
    return pl.pallas_call(
        _mid_body,
        grid=(_NG,),
        in_specs=[row, row, row, row, row, full, full, full, bias],
        out_specs=[row, row, row, row],
        out_shape=[nd, nd, nd, nd],
    )(r0, s1, s2, cin, cout, w1, w2, wr, br.reshape(1, _D))


def _fin_body(r_ref, s1_ref, s2_ref, cin_ref, cout_ref,
              x_ref, h1_ref, wfa_ref, wfb_ref, wfc_ref, bf_ref, y_ref):
    h2 = _finish(r_ref, s1_ref, s2_ref, cin_ref, cout_ref)
    y_ref[...] = (_dot(x_ref[...], wfa_ref[...])
                  + _dot(h1_ref[...], wfb_ref[...])
                  + _dot(h2, wfc_ref[...]) + bf_ref[...])


def _fin(r1, s1, s2, cin, cout, x, h1, wfa, wfb, wfc, bf):
    nd = jax.ShapeDtypeStruct((_N, _D), jnp.float32)
    row = pl.BlockSpec((_BN, _D), lambda i: (i, 0))
    full = pl.BlockSpec((_D, _D), lambda i: (0, 0))
    bias = pl.BlockSpec((1, _D), lambda i: (0, 0))
    return pl.pallas_call(
        _fin_body,
        grid=(_NG,),
        in_specs=[row, row, row, row, row, row, row, full, full, full, bias],
        out_specs=row,
        out_shape=nd,
    )(r1, s1, s2, cin, cout, x, h1, wfa, wfb, wfc, bf.reshape(1, _D))


def kernel(x, edge_index, W1_0, W2_0, Wr_0, br_0,
           W1_1, W2_1, Wr_1, br_1, Wf, bf):
    src = edge_index[0]
    dst = edge_index[1]
    src2d = src.reshape(_E // _CHUNK, _CHUNK)
    dst2d = dst.reshape(_E // _CHUNK, _CHUNK)
    z128 = jnp.zeros((_RZ, _D), jnp.float32)

    seg_cnt_k = _build_seg_kernel(True)
    seg_k = _build_seg_kernel(False)

    t1_0, t2_0, r0 = _mm3(x, W1_0, W2_0, Wr_0, br_0)
    s1_0, s2_0, cin, cout = seg_cnt_k(t1_0, t2_0, src2d, dst2d, z128,
                                      src, dst)
    cin = jnp.broadcast_to(cin.reshape(_NP, 1), (_NP, _D))
    cout = jnp.broadcast_to(cout.reshape(_NP, 1), (_NP, _D))

    h1, t1_1, t2_1, r1 = _mid(r0, s1_0, s2_0, cin, cout,
                              W1_1, W2_1, Wr_1, br_1)
    s1_1, s2_1 = seg_k(t1_1, t2_1, src2d, dst2d, z128)

    return _fin(r1, s1_1, s2_1, cin, cout, x, h1,
                Wf[:_D], Wf[_D:2 * _D], Wf[2 * _D:], bf)


# speedup vs baseline: 1.2544x; 1.2544x over previous
import functools

import jax
import jax.numpy as jnp
from jax import lax
from jax.experimental import pallas as pl
from jax.experimental.pallas import tpu as pltpu
from jax.experimental.pallas import tpu_sc as plsc

_N = 10000
_E = 320000
_D = 128

_NS = 16
_CHUNK = 125
_EPT = _E // _NS
_NCH = _EPT // _CHUNK
_BS = 32
_NB = _NCH // _BS
_NP = 10240
_RPT = _NP // _NS
_RZ = 128
_RCH = _RPT // _RZ
_CE = 2000
_NCS = _EPT // _CE
_CV = _CE // 16
_CR = _NP // _D

_BN = 1024
_NG = _NP // _BN


@functools.lru_cache(maxsize=None)
def _build_seg_kernel():
    mesh = plsc.VectorSubcoreMesh(core_axis_name="c", subcore_axis_name="s")

    @functools.partial(
        pl.kernel,
        out_type=[jax.ShapeDtypeStruct((_NP, _D), jnp.float32),
                  jax.ShapeDtypeStruct((_NP, _D), jnp.float32)],
        mesh=mesh,
        scratch_types=[
            pltpu.VMEM_SHARED((_NP, _D), jnp.float32),
            pltpu.VMEM((_BS, _CHUNK), jnp.int32),
            pltpu.VMEM((_BS, _CHUNK), jnp.int32),
            pltpu.VMEM((_RZ, _D), jnp.float32),
            pltpu.VMEM((_RZ, _D), jnp.float32),
            pltpu.SemaphoreType.DMA,
            pltpu.SemaphoreType.DMA,
        ],
    )
    def seg(t1, t2, src2d, dst2d, z128,
            s1_out, s2_out, acc, gidx, sidx, rowbuf, rowbuf1, sem, sem1):
        cid = lax.axis_index("c")
        tid = lax.axis_index("s")

        def one_pass(table, g2, s3, out):
            pltpu.sync_copy(z128, rowbuf)
            for z in range(_RCH):
                pltpu.sync_copy(
                    rowbuf, acc.at[pl.ds(tid * _RPT + z * _RZ, _RZ)])
            plsc.subcore_barrier()

            bufs = (rowbuf.at[pl.ds(0, _CHUNK)], rowbuf1.at[pl.ds(0, _CHUNK)])
            sems = (sem, sem1)

            def batch(b, carry):
                r = pl.multiple_of(tid * _NCH + b * _BS, _BS)
                pltpu.sync_copy(g2.at[pl.ds(r, _BS)], gidx)
                pltpu.sync_copy(s3.at[pl.ds(r, _BS)], sidx)
                descs = [None, None]
                descs[0] = pltpu.async_copy(table.at[gidx.at[0]], bufs[0],
                                            sems[0])
                for jj in range(_BS):
                    if jj + 1 < _BS:
                        nb = (jj + 1) % 2
                        descs[nb] = pltpu.async_copy(
                            table.at[gidx.at[jj + 1]], bufs[nb], sems[nb])
                    descs[jj % 2].wait()
                    pltpu.sync_copy(bufs[jj % 2], acc.at[sidx.at[jj]],
                                    add=True)
                return carry

            lax.fori_loop(0, _NB, batch, 0)
            plsc.subcore_barrier()
            for z in range(_RCH):
                r0 = tid * _RPT + z * _RZ
                pltpu.sync_copy(acc.at[pl.ds(r0, _RZ)],
                                out.at[pl.ds(r0, _RZ)])

        @pl.when(cid == 0)
        def _():
            one_pass(t1, src2d, dst2d, s1_out)

        @pl.when(cid == 1)
        def _():
            one_pass(t2, dst2d, src2d, s2_out)

    return seg


@functools.lru_cache(maxsize=None)
def _build_cnt_kernel():
    mesh = plsc.VectorSubcoreMesh(core_axis_name="c", subcore_axis_name="s")

    @functools.partial(
        pl.kernel,
        out_type=[jax.ShapeDtypeStruct((_CR, _D), jnp.float32),
                  jax.ShapeDtypeStruct((_CR, _D), jnp.float32)],
        mesh=mesh,
        scratch_types=[
            pltpu.VMEM_SHARED((_CR, _D), jnp.float32),
            pltpu.VMEM((_CE,), jnp.int32),
            pltpu.VMEM((_CE,), jnp.int32),
            pltpu.VMEM((_CR, _D), jnp.float32),
            pltpu.VMEM((_CR,), jnp.int32),
            pltpu.SemaphoreType.DMA,
            pltpu.SemaphoreType.DMA,
        ],
        compiler_params=pltpu.CompilerParams(needs_layout_passes=False),
    )
    def cnt(src1d, dst1d, z128,
            cin_out, cout_out, cacc, cidx0, cidx1, partial, ident, csem0,
            csem1):
        cid = lax.axis_index("c")
        tid = lax.axis_index("s")

        def one_pass(e1d, out):
            pltpu.sync_copy(z128.at[pl.ds(0, _CR)], partial)

            @pl.when(tid < 10)
            def _():
                pltpu.sync_copy(z128.at[pl.ds(0, 8)],
                                cacc.at[pl.ds(tid * 8, 8)])
            iota16 = lax.iota(jnp.int32, 16)
            for i in range(_CR // 16):
                ident[pl.ds(16 * i, 16)] = iota16 + 16 * i
            plsc.subcore_barrier()

            ones_v = jnp.ones((16,), jnp.float32)
            cbufs = (cidx0, cidx1)
            csems = (csem0, csem1)
            e0 = tid * _EPT

            def scan_buf(k):
                for v in range(_CV):
                    idx = cbufs[k][pl.ds(16 * v, 16)]
                    row = lax.shift_right_logical(idx, 7)
                    lane = idx & 127
                    plsc.addupdate_scatter(partial, [row, lane], ones_v)

            pltpu.async_copy(e1d.at[pl.ds(e0, _CE)], cidx0, csem0)

            def pair(p, carry):
                base = pl.multiple_of(e0 + p * (2 * _CE), 8)
                pltpu.async_copy(e1d.at[pl.ds(base + _CE, _CE)], cidx1,
                                 csem1)
                pltpu.make_async_copy(e1d.at[pl.ds(0, _CE)], cidx0,
                                      csem0).wait()
                scan_buf(0)
                nxt = pl.multiple_of(
                    jnp.minimum(base + 2 * _CE, _E - _CE), 8)
                pltpu.async_copy(e1d.at[pl.ds(nxt, _CE)], cidx0, csem0)
                pltpu.make_async_copy(e1d.at[pl.ds(0, _CE)], cidx1,
                                      csem1).wait()
                scan_buf(1)
                return carry

            lax.fori_loop(0, _NCS // 2, pair, 0)
            pltpu.make_async_copy(e1d.at[pl.ds(0, _CE)], cidx0, csem0).wait()
            pltpu.sync_copy(partial, cacc.at[ident], add=True)
            plsc.subcore_barrier()

            @pl.when(tid < 10)
            def _():
                pltpu.sync_copy(cacc.at[pl.ds(tid * 8, 8)],
                                out.at[pl.ds(tid * 8, 8)])

        @pl.when(cid == 0)
        def _():
            one_pass(dst1d, cin_out)

        @pl.when(cid == 1)
        def _():
            one_pass(src1d, cout_out)

    return cnt


def _dot(a, b):
    return jnp.dot(a, b, preferred_element_type=jnp.float32)


def _mm3_body(x_ref, w1_ref, w2_ref, wr_ref, br_ref, t1_ref, t2_ref, r_ref):
    xb = x_ref[...]
    t1_ref[...] = _dot(xb, w1_ref[...])
    t2_ref[...] = _dot(xb, w2_ref[...])
    r_ref[...] = _dot(xb, wr_ref[...]) + br_ref[...]


def _mm3(x, w1, w2, wr, br):
    nd = jax.ShapeDtypeStruct((_N, _D), jnp.float32)
    row = pl.BlockSpec((_BN, _D), lambda i: (i, 0))
    full = pl.BlockSpec((_D, _D), lambda i: (0, 0))
    bias = pl.BlockSpec((1, _D), lambda i: (0, 0))
    return pl.pallas_call(
        _mm3_body,
        grid=(_NG,),
        in_specs=[row, full, full, full, bias],
        out_specs=[row, row, row],
        out_shape=[nd, nd, nd],
    )(x, w1, w2, wr, br.reshape(1, _D))


def _finish(r_ref, s1_ref, s2_ref, cin_ref, cout_ref):
    rin = 1.0 / jnp.maximum(cin_ref[...], 1.0)
    rout = 1.0 / jnp.maximum(cout_ref[...], 1.0)
    return jnp.maximum(
        r_ref[...] + s1_ref[...] * rin + s2_ref[...] * rout, 0.0)


def _mid_body(r_ref, s1_ref, s2_ref, cin_ref, cout_ref,
              w1_ref, w2_ref, wr_ref, br_ref,
              h_ref, t1_ref, t2_ref, r1_ref):
    h = _finish(r_ref, s1_ref, s2_ref, cin_ref, cout_ref)
    h_ref[...] = h
    t1_ref[...] = _dot(h, w1_ref[...])
    t2_ref[...] = _dot(h, w2_ref[...])
    r1_ref[...] = _dot(h, wr_ref[...]) + br_ref[...]


def _mid(r0, s1, s2, cin, cout, w1, w2, wr, br):
    nd = jax.ShapeDtypeStruct((_N, _D), jnp.float32)
    row = pl.BlockSpec((_BN, _D), lambda i: (i, 0))
    full = pl.BlockSpec((_D, _D), lambda i: (0, 0))
    bias = pl.BlockSpec((1, _D), lambda i: (0, 0))
    return pl.pallas_call(
        _mid_body,
        grid=(_NG,),
        in_specs=[row, row, row, row, row, full, full, full, bias],
        out_specs=[row, row, row, row],
        out_shape=[nd, nd, nd, nd],
    )(r0, s1, s2, cin, cout, w1, w2, wr, br.reshape(1, _D))


def _fin_body(r_ref, s1_ref, s2_ref, cin_ref, cout_ref,
              x_ref, h1_ref, wfa_ref, wfb_ref, wfc_ref, bf_ref, y_ref):
    h2 = _finish(r_ref, s1_ref, s2_ref, cin_ref, cout_ref)
    y_ref[...] = (_dot(x_ref[...], wfa_ref[...])
                  + _dot(h1_ref[...], wfb_ref[...])
                  + _dot(h2, wfc_ref[...]) + bf_ref[...])


def _fin(r1, s1, s2, cin, cout, x, h1, wfa, wfb, wfc, bf):
    nd = jax.ShapeDtypeStruct((_N, _D), jnp.float32)
    row = pl.BlockSpec((_BN, _D), lambda i: (i, 0))
    full = pl.BlockSpec((_D, _D), lambda i: (0, 0))
    bias = pl.BlockSpec((1, _D), lambda i: (0, 0))
    return pl.pallas_call(
        _fin_body,
        grid=(_NG,),
        in_specs=[row, row, row, row, row, row, row, full, full, full, bias],
        out_specs=row,
        out_shape=nd,
    )(r1, s1, s2, cin, cout, x, h1, wfa, wfb, wfc, bf.reshape(1, _D))


def kernel(x, edge_index, W1_0, W2_0, Wr_0, br_0,
           W1_1, W2_1, Wr_1, br_1, Wf, bf):
    src = edge_index[0]
    dst = edge_index[1]
    src2d = src.reshape(_E // _CHUNK, _CHUNK)
    dst2d = dst.reshape(_E // _CHUNK, _CHUNK)
    z128 = jnp.zeros((_RZ, _D), jnp.float32)

    seg_k = _build_seg_kernel()
    cnt_k = _build_cnt_kernel()

    cin, cout = cnt_k(src, dst, z128)
    cin = jnp.broadcast_to(cin.reshape(_NP, 1), (_NP, _D))
    cout = jnp.broadcast_to(cout.reshape(_NP, 1), (_NP, _D))

    t1_0, t2_0, r0 = _mm3(x, W1_0, W2_0, Wr_0, br_0)
    s1_0, s2_0 = seg_k(t1_0, t2_0, src2d, dst2d, z128)

    h1, t1_1, t2_1, r1 = _mid(r0, s1_0, s2_0, cin, cout,
                              W1_1, W2_1, Wr_1, br_1)
    s1_1, s2_1 = seg_k(t1_1, t2_1, src2d, dst2d, z128)

    return _fin(r1, s1_1, s2_1, cin, cout, x, h1,
                Wf[:_D], Wf[_D:2 * _D], Wf[2 * _D:], bf)
